# Initial kernel scaffold; baseline (speedup 1.0000x reference)
import jax
import jax.numpy as jnp
from jax.experimental import pallas as pl


def kernel(x, orbitals_mf, orbitals_hf):
    raise NotImplementedError("write your pallas kernel here")


# speedup vs baseline: 1.8415x; 1.8415x over previous
import functools

import jax
import jax.numpy as jnp
from jax import lax
from jax.experimental import pallas as pl
from jax.experimental.pallas import tpu as pltpu
from jax.experimental.pallas import tpu_sc as plsc

_NUM_CORES = 2
_NUM_SUBCORES = 16
_LANES = 16
_IDX_MINOR = 128


def kernel(x, orbitals_mf, orbitals_hf):
    B, S = x.shape
    F = 2 * S
    D = orbitals_mf.shape[1] + orbitals_hf.shape[1]
    table = jnp.concatenate((orbitals_mf, orbitals_hf), axis=1)

    nw = _NUM_CORES * _NUM_SUBCORES
    bpw = B // nw
    n_chunks = F // _LANES
    n_half = S // _IDX_MINOR

    mesh = plsc.VectorSubcoreMesh(
        core_axis_name="c", subcore_axis_name="s",
        num_cores=_NUM_CORES, num_subcores=_NUM_SUBCORES)

    @functools.partial(
        pl.kernel,
        out_type=jax.ShapeDtypeStruct((B, S, D), jnp.float32),
        mesh=mesh,
        scratch_types=[
            pltpu.VMEM((S,), jnp.int32),
            pltpu.VMEM((n_half + 1, _IDX_MINOR), jnp.int32),
            pltpu.VMEM((S, D), jnp.float32),
            pltpu.SemaphoreType.DMA,
        ],
        compiler_params=pltpu.CompilerParams(
            use_tc_tiling_on_sc=False, needs_layout_passes=False),
    )
    def sc_kernel(x_hbm, table_hbm, out_hbm, x_v, idx_v, rows_v, sem):
        wid = lax.axis_index("s") * _NUM_CORES + lax.axis_index("c")
        base = wid * bpw
        iota = lax.iota(jnp.int32, _LANES)

        def body(bl, carry_unused):
            b = base + bl
            pltpu.sync_copy(x_hbm.at[b], x_v)

            carry = jnp.int32(0)
            for c in range(n_chunks):
                site = c if c < n_chunks // 2 else c - n_chunks // 2
                xi = x_v[pl.ds(site * _LANES, _LANES)]
                if c < n_chunks // 2:
                    mi = lax.shift_right_logical(xi + 1, 1)
                else:
                    mi = lax.shift_right_logical(1 - xi, 1)
                cs = plsc.cumsum(mi)
                pos = cs - mi + carry
                posf = mi * pos + (1 - mi) * (S + iota)
                row = lax.shift_right_logical(posf, 7)
                col = lax.bitwise_and(posf, _IDX_MINOR - 1)
                plsc.store_scatter(idx_v, [row, col], iota + c * _LANES)
                carry = carry + jnp.sum(mi)

            copies = []
            for h in range(n_half):
                copies.append(pltpu.async_copy(
                    table_hbm.at[idx_v.at[h]],
                    rows_v.at[pl.ds(h * _IDX_MINOR, _IDX_MINOR)],
                    sem))
            for cp in copies:
                cp.wait()
            pltpu.sync_copy(rows_v, out_hbm.at[b])
            return carry_unused

        lax.fori_loop(0, bpw, body, jnp.int32(0))

    return sc_kernel(x, table)
